# baseline (speedup 1.0000x reference)
import jax
import jax.numpy as jnp
from jax import lax
from jax.experimental import pallas as pl
from jax.experimental.pallas import tpu as pltpu
from jax.experimental.pallas import tpu_sc as plsc

NC = 2
NS = 16
LANES = 16

N = 10000
E = 320000
D = 128
H = 128
B = 128

CHUNK = 32
NFC = 312
NW = NC * NS
XTRA = (E - NW * NFC * CHUNK) // CHUNK
NROW = 8
NIDX = 12
UNROLL = 24
NA = 10240
ROWS_PER_TILE = NA // NS

_f32 = jnp.float32
_i32 = jnp.int32


def _agg_body(x_hbm, pack_hbm, parts_hbm, cnts_hbm,
              acc_sp, cnt_sp,
              rows0, rows1, rows2, rows3, rows4, rows5, rows6, rows7,
              ib0, ib1, ib2, ib3, ib4, ib5,
              ib6, ib7, ib8, ib9, ib10, ib11,
              ones_v,
              gsem0, gsem1, gsem2, gsem3, gsem4, gsem5, gsem6, gsem7,
              ssem0, ssem1, ssem2, ssem3, ssem4, ssem5, ssem6, ssem7,
              isem0, isem1, isem2, isem3, isem4, isem5,
              isem6, isem7, isem8, isem9, isem10, isem11,
              csem):
    c = lax.axis_index("c")
    s = lax.axis_index("s")
    wid = c * NS + s
    cbase = wid * NFC

    rows = (rows0, rows1, rows2, rows3, rows4, rows5, rows6, rows7)
    gsems = (gsem0, gsem1, gsem2, gsem3, gsem4, gsem5, gsem6, gsem7)
    ssems = (ssem0, ssem1, ssem2, ssem3, ssem4, ssem5, ssem6, ssem7)
    ibs = (ib0, ib1, ib2, ib3, ib4, ib5, ib6, ib7, ib8, ib9, ib10, ib11)
    isems = (isem0, isem1, isem2, isem3, isem4, isem5,
             isem6, isem7, isem8, isem9, isem10, isem11)

    def zrow(i, _):
        for j in range(D // LANES):
            rows0[i, pl.ds(LANES * j, LANES)] = jnp.zeros((LANES,), _f32)
        return 0
    lax.fori_loop(0, CHUNK, zrow, 0)
    for j in range(CHUNK // LANES):
        ones_v[pl.ds(LANES * j, LANES)] = jnp.ones((LANES,), _f32)

    NZ = ROWS_PER_TILE // CHUNK
    for k in range(NZ):
        pltpu.async_copy(rows0,
                         acc_sp.at[pl.ds(s * ROWS_PER_TILE + k * CHUNK, CHUNK)],
                         gsems[k % NROW])
    for k in range(ROWS_PER_TILE // D):
        pltpu.async_copy(rows0.at[0],
                         cnt_sp.at[pl.ds(s * ROWS_PER_TILE + k * D, D)],
                         ssems[k % NROW])
    for k in range(NZ):
        pltpu.make_async_copy(
            rows0, acc_sp.at[pl.ds(s * ROWS_PER_TILE + k * CHUNK, CHUNK)],
            gsems[k % NROW]).wait()
    for k in range(ROWS_PER_TILE // D):
        pltpu.make_async_copy(
            rows0.at[0], cnt_sp.at[pl.ds(s * ROWS_PER_TILE + k * D, D)],
            ssems[k % NROW]).wait()
    plsc.subcore_barrier()

    def load_idx(ci, slot):
        off = (cbase + ci) * (2 * CHUNK)
        pltpu.async_copy(pack_hbm.at[pl.ds(off, 2 * CHUNK)], ibs[slot],
                         isems[slot])

    def wait_idx(ci, slot):
        off = (cbase + ci) * (2 * CHUNK)
        pltpu.make_async_copy(pack_hbm.at[pl.ds(off, 2 * CHUNK)], ibs[slot],
                              isems[slot]).wait()

    def fire_gather(rs, isl):
        pltpu.async_copy(x_hbm.at[ibs[isl].at[pl.ds(0, CHUNK)]], rows[rs],
                         gsems[rs])

    def wait_gather(rs, isl):
        pltpu.make_async_copy(x_hbm.at[ibs[isl].at[pl.ds(0, CHUNK)]],
                              rows[rs], gsems[rs]).wait()

    def fire_scatter(rs, isl):
        dref = ibs[isl].at[pl.ds(CHUNK, CHUNK)]
        pltpu.async_copy(rows[rs], acc_sp.at[dref], ssems[rs], add=True)
        pltpu.async_copy(ones_v, cnt_sp.at[dref], csem, add=True)

    def wait_scatter(rs, isl):
        dref = ibs[isl].at[pl.ds(CHUNK, CHUNK)]
        pltpu.make_async_copy(rows[rs], acc_sp.at[dref], ssems[rs]).wait()
        pltpu.make_async_copy(ones_v, cnt_sp.at[dref], csem).wait()

    for k in range(8):
        load_idx(k, k)
    for k in range(4):
        wait_idx(k, k)
        fire_gather(k % NROW, k % NIDX)

    def step(iu, _):
        for u in range(UNROLL):
            ci = iu * UNROLL + u
            wait_gather(u % NROW, u % NIDX)
            fire_scatter(u % NROW, u % NIDX)

            @pl.when(ci >= 4)
            def _():
                wait_scatter((u - 4) % NROW, (u - 4) % NIDX)

            @pl.when(ci + 8 < NFC)
            def _():
                load_idx(ci + 8, (u + 8) % NIDX)

            @pl.when(ci + 4 < NFC)
            def _():
                wait_idx(ci + 4, (u + 4) % NIDX)
                fire_gather((u + 4) % NROW, (u + 4) % NIDX)
        return 0
    lax.fori_loop(0, NFC // UNROLL, step, 0)

    for ci in range(NFC - 4, NFC):
        wait_scatter(ci % NROW, ci % NIDX)

    @pl.when(wid < XTRA)
    def _():
        off = (NW * NFC + wid) * (2 * CHUNK)
        pltpu.sync_copy(pack_hbm.at[pl.ds(off, 2 * CHUNK)], ib0)
        pltpu.async_copy(x_hbm.at[ib0.at[pl.ds(0, CHUNK)]], rows0,
                         gsem0).wait()
        dref = ib0.at[pl.ds(CHUNK, CHUNK)]
        pltpu.sync_copy(rows0, acc_sp.at[dref], add=True)
        pltpu.sync_copy(ones_v, cnt_sp.at[dref], add=True)

    plsc.subcore_barrier()

    pltpu.sync_copy(acc_sp.at[pl.ds(s * ROWS_PER_TILE, ROWS_PER_TILE)],
                    parts_hbm.at[c, pl.ds(s * ROWS_PER_TILE, ROWS_PER_TILE)])
    pltpu.sync_copy(cnt_sp.at[pl.ds(s * ROWS_PER_TILE, ROWS_PER_TILE)],
                    cnts_hbm.at[pl.ds(c * NA + s * ROWS_PER_TILE,
                                      ROWS_PER_TILE)])


def _stage_a(x, pack):
    mesh = plsc.VectorSubcoreMesh(core_axis_name="c", subcore_axis_name="s",
                                  num_cores=NC, num_subcores=NS)
    idx = [pltpu.VMEM((2 * CHUNK,), _i32) for _ in range(NIDX)]
    return pl.kernel(
        _agg_body,
        out_type=(jax.ShapeDtypeStruct((NC, NA, D), _f32),
                  jax.ShapeDtypeStruct((NC * NA,), _f32)),
        mesh=mesh,
        scratch_types=(
            [pltpu.VMEM_SHARED((NA, D), _f32),
             pltpu.VMEM_SHARED((NA,), _f32)]
            + [pltpu.VMEM((CHUNK, D), _f32) for _ in range(NROW)]
            + idx
            + [pltpu.VMEM((CHUNK,), _f32)]
            + [pltpu.SemaphoreType.DMA] * (NROW + NROW + NIDX + 1)
        ),
        name="sage_edge_agg",
    )(x, pack)


def _mlp_body(p_ref, c_ref, x_ref, wl_ref, wr_ref, bl_ref, o_ref):
    p = p_ref[...]
    cm = c_ref[...]
    summed = p[0] + p[1]
    invc = 1.0 / jnp.maximum(cm[0, 0] + cm[1, 0], 1.0)
    dn_t = (((1,), (1,)), ((), ()))
    dn_n = (((1,), (0,)), ((), ()))
    ii = jnp.equal(lax.broadcasted_iota(_i32, (128, 128), 0),
                   lax.broadcasted_iota(_i32, (128, 128), 1))
    means = []
    nq = summed.shape[0] // 128
    for q in range(nq):
        dq = jnp.where(ii, invc[q][None, :], 0.0)
        sq = summed[q * 128:(q + 1) * 128]
        means.append(lax.dot_general(dq, sq, dn_n, preferred_element_type=_f32))
    mean = jnp.concatenate(means, axis=0)
    h = lax.dot_general(mean, wl_ref[...], dn_t, preferred_element_type=_f32)
    h = h + lax.dot_general(x_ref[...], wr_ref[...], dn_t,
                            preferred_element_type=_f32)
    h = h + bl_ref[...]
    o_ref[...] = jnp.maximum(h, 0.0)


def _stage_b(parts, cnts, x, W_l, W_r, b_l):
    RB = 512
    grid = (NA // RB,)
    return pl.pallas_call(
        _mlp_body,
        grid=grid,
        in_specs=[
            pl.BlockSpec((2, RB, D), lambda i: (0, i, 0)),
            pl.BlockSpec((2, 1, RB // 128, 128), lambda i: (0, i, 0, 0)),
            pl.BlockSpec((RB, D), lambda i: (i, 0)),
            pl.BlockSpec((H, D), lambda i: (0, 0)),
            pl.BlockSpec((H, D), lambda i: (0, 0)),
            pl.BlockSpec((1, H), lambda i: (0, 0)),
        ],
        out_specs=pl.BlockSpec((RB, H), lambda i: (i, 0)),
        out_shape=jax.ShapeDtypeStruct((NA, H), _f32),
        name="sage_mlp",
    )(parts, cnts.reshape(NC, NA // 512, 4, 128), x, W_l, W_r,
      b_l.reshape(1, H))


def _pool_body(h_hbm, batch_hbm, pooled_hbm, bbuf, cbuf, acc4, sem):
    del sem
    c = lax.axis_index("c")
    s = lax.axis_index("s")
    wid = c * NS + s
    g0 = wid * (B // (NC * NS))

    BCH = 2000
    carry = tuple(jnp.zeros((LANES,), _i32) for _ in range(5))
    for t in range(N // BCH):
        pltpu.sync_copy(batch_hbm.at[pl.ds(t * BCH, BCH)], bbuf)

        def cb(j, carry):
            v = bbuf[pl.ds(j * LANES, LANES)]
            outs = []
            for k in range(5):
                m = jnp.where(v < g0 + k, jnp.int32(1), jnp.int32(0))
                outs.append(carry[k] + m)
            return tuple(outs)
        carry = lax.fori_loop(0, BCH // LANES, cb, carry)

    bounds = []
    for k in range(5):
        vk = carry[k]
        ssum = vk[0]
        for l in range(1, LANES):
            ssum = ssum + vk[l]
        bounds.append(ssum)

    RCH = 64
    for k in range(4):
        start = bounds[k]
        end = bounds[k + 1]
        start8 = (start // 8) * 8
        nch = (end - start8 + (RCH - 1)) // RCH

        def chunk(cix, accs):
            off = pl.multiple_of(start8 + cix * RCH, 8)
            pltpu.sync_copy(h_hbm.at[pl.ds(off, RCH)], cbuf)

            def row(r, accs):
                pos = off + r
                valid = (pos >= start) & (pos < end)
                return tuple(
                    jnp.where(valid,
                              jnp.maximum(accs[j],
                                          cbuf[r, pl.ds(LANES * j, LANES)]),
                              accs[j])
                    for j in range(H // LANES))
            return lax.fori_loop(0, RCH, row, accs)

        accs = lax.fori_loop(0, nch, chunk,
                             tuple(jnp.zeros((LANES,), _f32)
                                   for _ in range(H // LANES)))
        for j in range(H // LANES):
            acc4[k, pl.ds(LANES * j, LANES)] = accs[j]

    pltpu.sync_copy(acc4, pooled_hbm.at[pl.ds(wid * 4, 4)])


def _stage_c(h, batch):
    mesh = plsc.VectorSubcoreMesh(core_axis_name="c", subcore_axis_name="s",
                                  num_cores=NC, num_subcores=NS)
    return pl.kernel(
        _pool_body,
        out_type=jax.ShapeDtypeStruct((B, H), _f32),
        mesh=mesh,
        scratch_types=[
            pltpu.VMEM((2000,), _i32),
            pltpu.VMEM((64, H), _f32),
            pltpu.VMEM((4, H), _f32),
            pltpu.SemaphoreType.DMA,
        ],
        name="sage_pool",
    )(h, batch)


def _head_body(p_ref, w4_ref, b4_ref, o_ref):
    dn = (((1,), (1,)), ((), ()))
    logits = lax.dot_general(p_ref[...], w4_ref[...], dn,
                             preferred_element_type=_f32)
    logits = logits + b4_ref[...]
    m = jnp.max(logits, axis=1, keepdims=True)
    z = logits - m
    lse = jnp.log(jnp.sum(jnp.exp(z), axis=1, keepdims=True))
    o_ref[...] = z - lse


def _stage_d(pooled, W4, b4):
    C = W4.shape[0]
    return pl.pallas_call(
        _head_body,
        out_shape=jax.ShapeDtypeStruct((B, C), _f32),
        name="sage_head",
    )(pooled, W4, b4.reshape(1, C))


@jax.jit
def kernel(x, edge_index, batch, embedding_data, W_l, b_l, W_r, W4, b4):
    del embedding_data

    pack = jnp.stack([edge_index[0].reshape(E // CHUNK, CHUNK),
                      edge_index[1].reshape(E // CHUNK, CHUNK)],
                     axis=1).reshape(-1)

    parts, cnts = _stage_a(x, pack)
    h = _stage_b(parts, cnts, x, W_l, W_r, b_l)
    pooled = _stage_c(h, batch)
    return _stage_d(pooled, W4, b4)


# speedup vs baseline: 1.4012x; 1.4012x over previous
import jax
import jax.numpy as jnp
from jax import lax
from jax.experimental import pallas as pl
from jax.experimental.pallas import tpu as pltpu
from jax.experimental.pallas import tpu_sc as plsc

NC = 2
NS = 16
LANES = 16

N = 10000
E = 320000
D = 128
H = 128
B = 128

CHUNK = 32
NFC = 312
NW = NC * NS
XTRA = (E - NW * NFC * CHUNK) // CHUNK
NROW = 8
NIDX = 12
UNROLL = 24
NA = 10240
ROWS_PER_TILE = NA // NS

_f32 = jnp.float32
_i32 = jnp.int32


def _agg_body(x_hbm, src_hbm, dst_hbm, parts_hbm, cnts_hbm,
              acc_sp, cnt_sp,
              rows0, rows1, rows2, rows3, rows4, rows5, rows6, rows7,
              sb0, sb1, sb2, sb3, sb4, sb5,
              sb6, sb7, sb8, sb9, sb10, sb11,
              db0, db1, db2, db3, db4, db5,
              db6, db7, db8, db9, db10, db11,
              ones_v,
              gsem0, gsem1, gsem2, gsem3, gsem4, gsem5, gsem6, gsem7,
              ssem0, ssem1, ssem2, ssem3, ssem4, ssem5, ssem6, ssem7,
              isem0, isem1, isem2, isem3, isem4, isem5,
              isem6, isem7, isem8, isem9, isem10, isem11,
              csem):
    c = lax.axis_index("c")
    s = lax.axis_index("s")
    wid = c * NS + s
    ebase = wid * (NFC * CHUNK)

    rows = (rows0, rows1, rows2, rows3, rows4, rows5, rows6, rows7)
    gsems = (gsem0, gsem1, gsem2, gsem3, gsem4, gsem5, gsem6, gsem7)
    ssems = (ssem0, ssem1, ssem2, ssem3, ssem4, ssem5, ssem6, ssem7)
    sbs = (sb0, sb1, sb2, sb3, sb4, sb5, sb6, sb7, sb8, sb9, sb10, sb11)
    dbs = (db0, db1, db2, db3, db4, db5, db6, db7, db8, db9, db10, db11)
    isems = (isem0, isem1, isem2, isem3, isem4, isem5,
             isem6, isem7, isem8, isem9, isem10, isem11)

    def zrow(i, _):
        for j in range(D // LANES):
            rows0[i, pl.ds(LANES * j, LANES)] = jnp.zeros((LANES,), _f32)
        return 0
    lax.fori_loop(0, CHUNK, zrow, 0)
    for j in range(CHUNK // LANES):
        ones_v[pl.ds(LANES * j, LANES)] = jnp.ones((LANES,), _f32)

    NZ = ROWS_PER_TILE // CHUNK
    for k in range(NZ):
        pltpu.async_copy(rows0,
                         acc_sp.at[pl.ds(s * ROWS_PER_TILE + k * CHUNK, CHUNK)],
                         gsems[k % NROW])
    for k in range(ROWS_PER_TILE // D):
        pltpu.async_copy(rows0.at[0],
                         cnt_sp.at[pl.ds(s * ROWS_PER_TILE + k * D, D)],
                         ssems[k % NROW])
    for k in range(NZ):
        pltpu.make_async_copy(
            rows0, acc_sp.at[pl.ds(s * ROWS_PER_TILE + k * CHUNK, CHUNK)],
            gsems[k % NROW]).wait()
    for k in range(ROWS_PER_TILE // D):
        pltpu.make_async_copy(
            rows0.at[0], cnt_sp.at[pl.ds(s * ROWS_PER_TILE + k * D, D)],
            ssems[k % NROW]).wait()
    plsc.subcore_barrier()

    def load_idx(ci, slot):
        off = ebase + ci * CHUNK
        pltpu.async_copy(src_hbm.at[pl.ds(off, CHUNK)], sbs[slot], isems[slot])
        pltpu.async_copy(dst_hbm.at[pl.ds(off, CHUNK)], dbs[slot], isems[slot])

    def wait_idx(ci, slot):
        off = ebase + ci * CHUNK
        pltpu.make_async_copy(src_hbm.at[pl.ds(off, CHUNK)], sbs[slot],
                              isems[slot]).wait()
        pltpu.make_async_copy(dst_hbm.at[pl.ds(off, CHUNK)], dbs[slot],
                              isems[slot]).wait()

    def fire_gather(rs, isl):
        pltpu.async_copy(x_hbm.at[sbs[isl]], rows[rs], gsems[rs])

    def wait_gather(rs, isl):
        pltpu.make_async_copy(x_hbm.at[sbs[isl]], rows[rs],
                              gsems[rs]).wait()

    def fire_scatter(rs, isl):
        pltpu.async_copy(rows[rs], acc_sp.at[dbs[isl]], ssems[rs], add=True)
        pltpu.async_copy(ones_v, cnt_sp.at[dbs[isl]], csem, add=True)

    def wait_scatter(rs, isl):
        pltpu.make_async_copy(rows[rs], acc_sp.at[dbs[isl]],
                              ssems[rs]).wait()
        pltpu.make_async_copy(ones_v, cnt_sp.at[dbs[isl]], csem).wait()

    for k in range(8):
        load_idx(k, k)
    for k in range(4):
        wait_idx(k, k)
        fire_gather(k % NROW, k % NIDX)

    def step(iu, _):
        for u in range(UNROLL):
            ci = iu * UNROLL + u
            wait_gather(u % NROW, u % NIDX)
            fire_scatter(u % NROW, u % NIDX)

            @pl.when(ci >= 4)
            def _():
                wait_scatter((u - 4) % NROW, (u - 4) % NIDX)

            @pl.when(ci + 8 < NFC)
            def _():
                load_idx(ci + 8, (u + 8) % NIDX)

            @pl.when(ci + 4 < NFC)
            def _():
                wait_idx(ci + 4, (u + 4) % NIDX)
                fire_gather((u + 4) % NROW, (u + 4) % NIDX)
        return 0
    lax.fori_loop(0, NFC // UNROLL, step, 0)

    for ci in range(NFC - 4, NFC):
        wait_scatter(ci % NROW, ci % NIDX)

    @pl.when(wid < XTRA)
    def _():
        off = NW * NFC * CHUNK + wid * CHUNK
        pltpu.sync_copy(src_hbm.at[pl.ds(off, CHUNK)], sb0)
        pltpu.sync_copy(dst_hbm.at[pl.ds(off, CHUNK)], db0)
        pltpu.async_copy(x_hbm.at[sb0], rows0, gsem0).wait()
        pltpu.sync_copy(rows0, acc_sp.at[db0], add=True)
        pltpu.sync_copy(ones_v, cnt_sp.at[db0], add=True)

    plsc.subcore_barrier()

    pltpu.sync_copy(acc_sp.at[pl.ds(s * ROWS_PER_TILE, ROWS_PER_TILE)],
                    parts_hbm.at[c, pl.ds(s * ROWS_PER_TILE, ROWS_PER_TILE)])
    pltpu.sync_copy(cnt_sp.at[pl.ds(s * ROWS_PER_TILE, ROWS_PER_TILE)],
                    cnts_hbm.at[pl.ds(c * NA + s * ROWS_PER_TILE,
                                      ROWS_PER_TILE)])


def _stage_a(x, src, dst):
    mesh = plsc.VectorSubcoreMesh(core_axis_name="c", subcore_axis_name="s",
                                  num_cores=NC, num_subcores=NS)
    idx = [pltpu.VMEM((CHUNK,), _i32) for _ in range(2 * NIDX)]
    return pl.kernel(
        _agg_body,
        out_type=(jax.ShapeDtypeStruct((NC, NA, D), _f32),
                  jax.ShapeDtypeStruct((NC * NA,), _f32)),
        mesh=mesh,
        scratch_types=(
            [pltpu.VMEM_SHARED((NA, D), _f32),
             pltpu.VMEM_SHARED((NA,), _f32)]
            + [pltpu.VMEM((CHUNK, D), _f32) for _ in range(NROW)]
            + idx
            + [pltpu.VMEM((CHUNK,), _f32)]
            + [pltpu.SemaphoreType.DMA] * (NROW + NROW + NIDX + 1)
        ),
        name="sage_edge_agg",
    )(x, src, dst)


def _mlp_body(p_ref, c_ref, x_ref, wl_ref, wr_ref, bl_ref, o_ref):
    p = p_ref[...]
    cm = c_ref[...]
    summed = p[0] + p[1]
    invc = 1.0 / jnp.maximum(cm[0, 0] + cm[1, 0], 1.0)
    dn_t = (((1,), (1,)), ((), ()))
    dn_n = (((1,), (0,)), ((), ()))
    ii = jnp.equal(lax.broadcasted_iota(_i32, (128, 128), 0),
                   lax.broadcasted_iota(_i32, (128, 128), 1))
    means = []
    nq = summed.shape[0] // 128
    for q in range(nq):
        dq = jnp.where(ii, invc[q][None, :], 0.0)
        sq = summed[q * 128:(q + 1) * 128]
        means.append(lax.dot_general(dq, sq, dn_n, preferred_element_type=_f32))
    mean = jnp.concatenate(means, axis=0)
    h = lax.dot_general(mean, wl_ref[...], dn_t, preferred_element_type=_f32)
    h = h + lax.dot_general(x_ref[...], wr_ref[...], dn_t,
                            preferred_element_type=_f32)
    h = h + bl_ref[...]
    o_ref[...] = jnp.maximum(h, 0.0)


def _stage_b(parts, cnts, x, W_l, W_r, b_l):
    RB = 512
    grid = (NA // RB,)
    return pl.pallas_call(
        _mlp_body,
        grid=grid,
        in_specs=[
            pl.BlockSpec((2, RB, D), lambda i: (0, i, 0)),
            pl.BlockSpec((2, 1, RB // 128, 128), lambda i: (0, i, 0, 0)),
            pl.BlockSpec((RB, D), lambda i: (i, 0)),
            pl.BlockSpec((H, D), lambda i: (0, 0)),
            pl.BlockSpec((H, D), lambda i: (0, 0)),
            pl.BlockSpec((1, H), lambda i: (0, 0)),
        ],
        out_specs=pl.BlockSpec((RB, H), lambda i: (i, 0)),
        out_shape=jax.ShapeDtypeStruct((NA, H), _f32),
        name="sage_mlp",
    )(parts, cnts.reshape(NC, NA // 512, 4, 128), x, W_l, W_r,
      b_l.reshape(1, H))


RCH = 64
NBLK = N // LANES


def _pool_body(h_hbm, batch_hbm, pooled_hbm, bbuf, cb0, cb1, acc4,
               sem0, sem1):
    c = lax.axis_index("c")
    s = lax.axis_index("s")
    wid = c * NS + s
    g0 = wid * (B // (NC * NS))

    pltpu.sync_copy(batch_hbm, bbuf.at[pl.ds(0, N)])
    bbuf[pl.ds(N, LANES)] = jnp.full((LANES,), B, _i32)

    def lower_bound(g):
        def bs(_, lohi):
            lo, hi = lohi
            mid = (lo + hi) // 2
            v = bbuf[pl.ds(mid * LANES, LANES)]
            big = v[0] >= g
            return (jnp.where(big, lo, mid + 1), jnp.where(big, mid, hi))
        lo, _ = lax.fori_loop(0, 10, bs, (jnp.int32(0), jnp.int32(NBLK)))
        bm1 = jnp.maximum(lo - 1, 0)
        v = bbuf[pl.ds(bm1 * LANES, LANES)]
        m = jnp.where(v < g, jnp.int32(1), jnp.int32(0))
        csum = m[0]
        for l in range(1, LANES):
            csum = csum + m[l]
        return jnp.where(lo == 0, jnp.int32(0), (lo - 1) * LANES + csum)

    bounds = [lower_bound(g0 + k) for k in range(5)]

    def fire(off, cb, sem):
        pltpu.async_copy(h_hbm.at[pl.ds(pl.multiple_of(off, 8), RCH)],
                         cb, sem)

    def wait(off, cb, sem):
        pltpu.make_async_copy(h_hbm.at[pl.ds(pl.multiple_of(off, 8), RCH)],
                              cb, sem).wait()

    def reduce_chunk(cb, off, start, end, accs):
        lo_r = jnp.maximum(start - off, 0)
        hi_r = jnp.minimum(end - off, RCH)

        def row(r, accs):
            return tuple(jnp.maximum(accs[j], cb[r, pl.ds(LANES * j, LANES)])
                         for j in range(H // LANES))
        return lax.fori_loop(lo_r, hi_r, row, accs)

    for k in range(4):
        start = bounds[k]
        end = bounds[k + 1]
        start8 = (start // 8) * 8
        nch = (end - start8 + (RCH - 1)) // RCH

        @pl.when(nch > 0)
        def _():
            fire(start8, cb0, sem0)

        accs0 = tuple(jnp.zeros((LANES,), _f32) for _ in range(H // LANES))

        def pair(ip, accs):
            e = 2 * ip
            o = e + 1
            off_e = start8 + e * RCH
            off_o = start8 + o * RCH

            @pl.when(o < nch)
            def _():
                fire(off_o, cb1, sem1)

            @pl.when(e < nch)
            def _():
                wait(off_e, cb0, sem0)
            accs = reduce_chunk(cb0, off_e, start, end, accs)

            @pl.when(o + 1 < nch)
            def _():
                fire(off_o + RCH, cb0, sem0)

            @pl.when(o < nch)
            def _():
                wait(off_o, cb1, sem1)
            accs = reduce_chunk(cb1, off_o, start, end, accs)
            return accs

        accs = lax.fori_loop(0, (nch + 1) // 2, pair, accs0)
        for j in range(H // LANES):
            acc4[k, pl.ds(LANES * j, LANES)] = accs[j]

    pltpu.sync_copy(acc4, pooled_hbm.at[pl.ds(wid * 4, 4)])


def _stage_c(h, batch):
    mesh = plsc.VectorSubcoreMesh(core_axis_name="c", subcore_axis_name="s",
                                  num_cores=NC, num_subcores=NS)
    return pl.kernel(
        _pool_body,
        out_type=jax.ShapeDtypeStruct((B, H), _f32),
        mesh=mesh,
        scratch_types=[
            pltpu.VMEM((N + LANES,), _i32),
            pltpu.VMEM((RCH, H), _f32),
            pltpu.VMEM((RCH, H), _f32),
            pltpu.VMEM((4, H), _f32),
            pltpu.SemaphoreType.DMA,
            pltpu.SemaphoreType.DMA,
        ],
        name="sage_pool",
    )(h, batch)


def _head_body(p_ref, w4_ref, b4_ref, o_ref):
    dn = (((1,), (1,)), ((), ()))
    logits = lax.dot_general(p_ref[...], w4_ref[...], dn,
                             preferred_element_type=_f32)
    logits = logits + b4_ref[...]
    m = jnp.max(logits, axis=1, keepdims=True)
    z = logits - m
    lse = jnp.log(jnp.sum(jnp.exp(z), axis=1, keepdims=True))
    o_ref[...] = z - lse


def _stage_d(pooled, W4, b4):
    C = W4.shape[0]
    return pl.pallas_call(
        _head_body,
        out_shape=jax.ShapeDtypeStruct((B, C), _f32),
        name="sage_head",
    )(pooled, W4, b4.reshape(1, C))


@jax.jit
def kernel(x, edge_index, batch, embedding_data, W_l, b_l, W_r, W4, b4):
    del embedding_data

    parts, cnts = _stage_a(x, edge_index[0], edge_index[1])
    h = _stage_b(parts, cnts, x, W_l, W_r, b_l)
    pooled = _stage_c(h, batch)
    return _stage_d(pooled, W4, b4)
